# baseline (speedup 1.0000x reference)
import jax
import jax.numpy as jnp
from jax import lax
from jax.experimental import pallas as pl
from jax.experimental.pallas import tpu as pltpu
from jax.experimental.pallas import tpu_sc as plsc

E = 300
EP = 384
NC, NS = 2, 16
NW = NC * NS
CQ = 96
CD = 80
CM = max(CQ, CD)


NSLOT = 3


def _sc_body(q_idx, d_idx, qt, dt, q_out, d_out, *scratch):
    wid = lax.axis_index("s") * NC + lax.axis_index("c")
    idx = scratch[0:NSLOT]
    rows = scratch[NSLOT:2 * NSLOT]
    gsem = scratch[2 * NSLOT:3 * NSLOT]
    wsem = scratch[3 * NSLOT:4 * NSLOT]
    isem = scratch[4 * NSLOT:5 * NSLOT]

    def phase(idx_hbm, table, out_hbm, c):
        per_w = idx_hbm.shape[0] // NW
        n = per_w // c
        base_w = wid * per_w

        def idx_load(s, i):
            pltpu.async_copy(idx_hbm.at[pl.ds(base_w + i * c, c)],
                             idx[s].at[pl.ds(0, c)], isem[s])

        def idx_wait(s):
            pltpu.make_async_copy(idx_hbm.at[pl.ds(0, c)],
                                  idx[s].at[pl.ds(0, c)], isem[s]).wait()

        def gather_start(s):
            pltpu.async_copy(table.at[idx[s].at[pl.ds(0, c)]],
                             rows[s].at[pl.ds(0, c)], gsem[s])

        def gather_wait(s):
            pltpu.make_async_copy(table.at[idx[s].at[pl.ds(0, c)]],
                                  rows[s].at[pl.ds(0, c)], gsem[s]).wait()

        def wb_start(s, i):
            pltpu.async_copy(rows[s].at[pl.ds(0, c)],
                             out_hbm.at[pl.ds(base_w + i * c, c)], wsem[s])

        def wb_wait(s):
            pltpu.make_async_copy(rows[s].at[pl.ds(0, c)],
                                  out_hbm.at[pl.ds(0, c)], wsem[s]).wait()

        for j in range(min(NSLOT, n)):
            idx_load(j, j)
        idx_wait(0)
        gather_start(0)
        idx_wait(1)
        gather_start(1)

        body = n - n % NSLOT

        @pl.loop(0, body, step=NSLOT)
        def _(g):
            for s in range(NSLOT):
                i = g + s
                nxt = (s + 2) % NSLOT

                @pl.when(i + 2 < n)
                def _():
                    @pl.when(i >= 1)
                    def _():
                        wb_wait(nxt)
                    idx_wait(nxt)
                    gather_start(nxt)

                gather_wait(s)
                wb_start(s, i)

                @pl.when(i + 3 < n)
                def _():
                    idx_load(s, i + 3)

        for i in range(body, n):
            s = i % NSLOT
            nxt = (s + 2) % NSLOT
            if i + 2 < n:
                if i >= 1:
                    wb_wait(nxt)
                idx_wait(nxt)
                gather_start(nxt)
            gather_wait(s)
            wb_start(s, i)
            if i + 3 < n:
                idx_load(s, i + 3)

        for s in range(NSLOT):
            wb_wait(s)

    phase(q_idx, qt, q_out, CQ)
    phase(d_idx, dt, d_out, CD)


def kernel(query, doc, query_table, doc_table):
    B, Lq = query.shape
    _, Ld = doc.shape
    V = query_table.shape[0]
    LqP = -(-Lq // 8) * 8
    pad_block = (jnp.arange(B * (LqP - Lq), dtype=jnp.int32) % V).reshape(
        B, LqP - Lq)
    q_idx = jnp.concatenate(
        [query.astype(jnp.int32), pad_block], axis=1).reshape(-1)
    d_idx = doc.reshape(-1).astype(jnp.int32)
    qt = jnp.pad(query_table, ((0, 0), (0, EP - E)))
    dt = jnp.pad(doc_table, ((0, 0), (0, EP - E)))

    call = pl.kernel(
        _sc_body,
        out_type=(
            jax.ShapeDtypeStruct((q_idx.shape[0], EP), jnp.float32),
            jax.ShapeDtypeStruct((d_idx.shape[0], EP), jnp.float32),
        ),
        mesh=plsc.VectorSubcoreMesh(
            core_axis_name="c", subcore_axis_name="s",
            num_cores=NC, num_subcores=NS,
        ),
        scratch_types=(
            [pltpu.VMEM((CM,), jnp.int32)] * NSLOT
            + [pltpu.VMEM((CM, EP), jnp.float32)] * NSLOT
            + [pltpu.SemaphoreType.DMA] * (3 * NSLOT)
        ),
    )
    q_rows, d_rows = call(q_idx, d_idx, qt, dt)
    return (q_rows.reshape(B, LqP, EP)[:, :Lq, :E],
            d_rows[:, :E].reshape(B, Ld, E))
